# baseline (speedup 1.0000x reference)
import functools

import jax
import jax.numpy as jnp
from jax import lax
from jax.experimental import pallas as pl
from jax.experimental.pallas import tpu as pltpu
from jax.experimental.pallas import tpu_sc as plsc

_COMMIT = 0.25
_BN = 256


def _vq_tc_kernel(x_ref, w_ref,
                  dist_ref, oh_ref, idx_ref, loss_ref, perp_ref,
                  counts_ref, ws_ref, acc_ref):
    g = pl.program_id(0)
    i = pl.program_id(1)
    ng = pl.num_programs(0)
    nb = pl.num_programs(1)

    x = x_ref[0]
    w = w_ref[0]
    bn, d = x.shape
    k = w.shape[0]

    @pl.when(jnp.logical_and(g == 0, i == 0))
    def _():
        acc_ref[0] = 0.0
        acc_ref[1] = 0.0

    @pl.when(i == 0)
    def _():
        counts_ref[...] = jnp.zeros_like(counts_ref)
        w2 = w * w
        ones = jnp.ones((1, d), jnp.float32)
        ws_ref[...] = jax.lax.dot_general(
            ones, w2, (((1,), (1,)), ((), ())),
            preferred_element_type=jnp.float32)

    xs = jnp.sum(x * x, axis=1, keepdims=True)
    dots = jax.lax.dot_general(
        x, w, (((1,), (1,)), ((), ())),
        preferred_element_type=jnp.float32)
    dist = xs + ws_ref[...] - 2.0 * dots
    dist_ref[0] = dist

    mind = jnp.min(dist, axis=1, keepdims=True)
    iota = jax.lax.broadcasted_iota(jnp.int32, (bn, k), 1)
    idx = jnp.min(jnp.where(dist == mind, iota, k),
                  axis=1, keepdims=True)
    oh = (iota == idx).astype(jnp.float32)
    oh_ref[0] = oh
    idx_ref[0] = idx + g * k
    counts_ref[...] += jnp.sum(oh, axis=0, keepdims=True)

    acc_ref[0] += jnp.sum(mind)

    @pl.when(i == nb - 1)
    def _():
        n_tok = nb * bn
        p = counts_ref[...] * (1.0 / n_tok)
        ent = -jnp.sum(p * jnp.log(p + 1e-10))
        acc_ref[1] += jnp.exp(ent)

    @pl.when(jnp.logical_and(g == ng - 1, i == nb - 1))
    def _():
        n_tok = nb * bn
        loss_ref[...] = jnp.full(
            (1, 1), acc_ref[0] * ((1.0 + _COMMIT) / (ng * n_tok * d)),
            dtype=jnp.float32)
        perp_ref[...] = jnp.full((1, 1), acc_ref[1] * (1.0 / ng),
                                 dtype=jnp.float32)


def _sc_gather(table_sup, idx_flat, d_out):
    b, = idx_flat.shape
    _, dp = table_sup.shape
    info = plsc.get_sparse_core_info()
    nl = info.num_lanes
    nw = info.num_cores * info.num_subcores
    bpw = b // nw
    mesh = plsc.VectorSubcoreMesh(core_axis_name="c", subcore_axis_name="s")

    @functools.partial(
        pl.kernel, mesh=mesh,
        out_type=jax.ShapeDtypeStruct((b, dp), jnp.float32),
        scratch_types=[
            pltpu.VMEM((bpw,), jnp.int32),
            pltpu.VMEM((bpw,), jnp.int32),
            pltpu.VMEM((bpw, dp), jnp.float32),
            pltpu.SemaphoreType.DMA,
        ],
    )
    def k(idx_hbm, table_hbm, out_hbm, idx_v, sup_v, rows_v, sem):
        wid = lax.axis_index("s") * info.num_cores + lax.axis_index("c")
        base = wid * bpw
        pltpu.sync_copy(idx_hbm.at[pl.ds(base, bpw)], idx_v)
        for m in range(bpw // nl):
            iv = idx_v[pl.ds(m * nl, nl)]
            sup_v[pl.ds(m * nl, nl)] = lax.shift_right_logical(iv, 2)
        pltpu.async_copy(table_hbm.at[sup_v], rows_v, sem).wait()
        pltpu.sync_copy(rows_v, out_hbm.at[pl.ds(base, bpw)])

    return k(idx_flat, table_sup)


def kernel(inputs, weights):
    n, g, d = inputs.shape
    _, k, _ = weights.shape
    bn = _BN
    x = jnp.transpose(inputs, (1, 0, 2))

    grid = (g, n // bn)
    out_shape = (
        jax.ShapeDtypeStruct((g, n, k), jnp.float32),
        jax.ShapeDtypeStruct((g, n, k), jnp.float32),
        jax.ShapeDtypeStruct((g, n, 1), jnp.int32),
        jax.ShapeDtypeStruct((1, 1), jnp.float32),
        jax.ShapeDtypeStruct((1, 1), jnp.float32),
    )
    in_specs = [
        pl.BlockSpec((1, bn, d), lambda gi, ii: (gi, ii, 0)),
        pl.BlockSpec((1, k, d), lambda gi, ii: (gi, 0, 0)),
    ]
    out_specs = (
        pl.BlockSpec((1, bn, k), lambda gi, ii: (gi, ii, 0)),
        pl.BlockSpec((1, bn, k), lambda gi, ii: (gi, ii, 0)),
        pl.BlockSpec((1, bn, 1), lambda gi, ii: (gi, ii, 0)),
        pl.BlockSpec((1, 1), lambda gi, ii: (0, 0)),
        pl.BlockSpec((1, 1), lambda gi, ii: (0, 0)),
    )
    scratch_shapes = [
        pltpu.VMEM((1, k), jnp.float32),
        pltpu.VMEM((1, k), jnp.float32),
        pltpu.SMEM((2,), jnp.float32),
    ]
    dist, oh, idx, loss, perp = pl.pallas_call(
        _vq_tc_kernel,
        grid=grid,
        in_specs=in_specs,
        out_specs=out_specs,
        out_shape=out_shape,
        scratch_shapes=scratch_shapes,
    )(x, weights)

    table_sup = weights.reshape(g * k // 4, 4 * d)
    idx_flat = idx.reshape(g * n)
    sup_rows = _sc_gather(table_sup, idx_flat, d).reshape(g * n, 4, d)
    emb = jnp.take_along_axis(
        sup_rows, (idx_flat & 3)[:, None, None], axis=1)[:, 0].reshape(g, n, d)
    quantized_all = jnp.transpose(emb, (1, 0, 2))
    return (loss[0, 0], quantized_all, perp[0, 0], emb, oh, dist)


# speedup vs baseline: 1.0842x; 1.0842x over previous
import functools

import jax
import jax.numpy as jnp
from jax import lax
from jax.experimental import pallas as pl
from jax.experimental.pallas import tpu as pltpu
from jax.experimental.pallas import tpu_sc as plsc

_COMMIT = 0.25
_BN = 256


def _vq_tc_kernel(x_ref, w_ref,
                  dist_ref, oh_ref, idx_ref, loss_ref, perp_ref,
                  counts_ref, ws_ref, acc_ref):
    g = pl.program_id(0)
    i = pl.program_id(1)
    ng = pl.num_programs(0)
    nb = pl.num_programs(1)

    x = x_ref[0]
    w = w_ref[0]
    bn, d = x.shape
    k = w.shape[0]

    @pl.when(jnp.logical_and(g == 0, i == 0))
    def _():
        acc_ref[0] = 0.0
        acc_ref[1] = 0.0

    @pl.when(i == 0)
    def _():
        counts_ref[...] = jnp.zeros_like(counts_ref)
        w2 = w * w
        ones = jnp.ones((1, d), jnp.float32)
        ws_ref[...] = jax.lax.dot_general(
            ones, w2, (((1,), (1,)), ((), ())),
            preferred_element_type=jnp.float32)

    xs = jnp.sum(x * x, axis=1, keepdims=True)
    dots = jax.lax.dot_general(
        x, w, (((1,), (1,)), ((), ())),
        preferred_element_type=jnp.float32)
    dist = xs + ws_ref[...] - 2.0 * dots
    dist_ref[0] = dist

    mind = jnp.min(dist, axis=1, keepdims=True)
    iota = jax.lax.broadcasted_iota(jnp.int32, (bn, k), 1)
    idx = jnp.min(jnp.where(dist == mind, iota, k),
                  axis=1, keepdims=True)
    oh = (iota == idx).astype(jnp.float32)
    oh_ref[0] = oh
    idx_ref[0] = idx + g * k
    counts_ref[...] += jnp.sum(oh, axis=0, keepdims=True)

    acc_ref[0] += jnp.sum(mind)

    @pl.when(i == nb - 1)
    def _():
        n_tok = nb * bn
        p = counts_ref[...] * (1.0 / n_tok)
        ent = -jnp.sum(p * jnp.log(p + 1e-10))
        acc_ref[1] += jnp.exp(ent)

    @pl.when(jnp.logical_and(g == ng - 1, i == nb - 1))
    def _():
        n_tok = nb * bn
        loss_ref[...] = jnp.full(
            (1, 1), acc_ref[0] * ((1.0 + _COMMIT) / (ng * n_tok * d)),
            dtype=jnp.float32)
        perp_ref[...] = jnp.full((1, 1), acc_ref[1] * (1.0 / ng),
                                 dtype=jnp.float32)


def _sc_gather(table_sup, idx_flat, d_out):
    b, = idx_flat.shape
    _, dp = table_sup.shape
    info = plsc.get_sparse_core_info()
    nl = info.num_lanes
    nw = info.num_cores * info.num_subcores
    bpw = b // nw
    mesh = plsc.VectorSubcoreMesh(core_axis_name="c", subcore_axis_name="s")

    @functools.partial(
        pl.kernel, mesh=mesh,
        out_type=jax.ShapeDtypeStruct((b, dp), jnp.float32),
        scratch_types=[
            pltpu.VMEM((bpw,), jnp.int32),
            pltpu.VMEM((bpw,), jnp.int32),
            pltpu.VMEM((bpw, dp), jnp.float32),
            pltpu.SemaphoreType.DMA,
        ],
    )
    def k(idx_hbm, table_hbm, out_hbm, idx_v, sup_v, rows_v, sem):
        wid = lax.axis_index("s") * info.num_cores + lax.axis_index("c")
        base = wid * bpw
        pltpu.sync_copy(idx_hbm.at[pl.ds(base, bpw)], idx_v)
        for m in range(bpw // nl):
            iv = idx_v[pl.ds(m * nl, nl)]
            sup_v[pl.ds(m * nl, nl)] = lax.shift_right_logical(iv, 2)
        pltpu.async_copy(table_hbm.at[sup_v], rows_v, sem).wait()
        pltpu.sync_copy(rows_v, out_hbm.at[pl.ds(base, bpw)])

    return k(idx_flat, table_sup)


def kernel(inputs, weights):
    n, g, d = inputs.shape
    _, k, _ = weights.shape
    bn = _BN
    x = jnp.transpose(inputs, (1, 0, 2))

    grid = (g, n // bn)
    out_shape = (
        jax.ShapeDtypeStruct((g, n, k), jnp.float32),
        jax.ShapeDtypeStruct((g, n, k), jnp.float32),
        jax.ShapeDtypeStruct((g, n, 1), jnp.int32),
        jax.ShapeDtypeStruct((1, 1), jnp.float32),
        jax.ShapeDtypeStruct((1, 1), jnp.float32),
    )
    in_specs = [
        pl.BlockSpec((1, bn, d), lambda gi, ii: (gi, ii, 0)),
        pl.BlockSpec((1, k, d), lambda gi, ii: (gi, 0, 0)),
    ]
    out_specs = (
        pl.BlockSpec((1, bn, k), lambda gi, ii: (gi, ii, 0)),
        pl.BlockSpec((1, bn, k), lambda gi, ii: (gi, ii, 0)),
        pl.BlockSpec((1, bn, 1), lambda gi, ii: (gi, ii, 0)),
        pl.BlockSpec((1, 1), lambda gi, ii: (0, 0)),
        pl.BlockSpec((1, 1), lambda gi, ii: (0, 0)),
    )
    scratch_shapes = [
        pltpu.VMEM((1, k), jnp.float32),
        pltpu.VMEM((1, k), jnp.float32),
        pltpu.SMEM((2,), jnp.float32),
    ]
    dist, oh, idx, loss, perp = pl.pallas_call(
        _vq_tc_kernel,
        grid=grid,
        in_specs=in_specs,
        out_specs=out_specs,
        out_shape=out_shape,
        scratch_shapes=scratch_shapes,
    )(x, weights)

    table_sup = weights.reshape(g * k // 4, 4 * d)
    idx_flat = idx.reshape(g * n)
    sup_rows = _sc_gather(table_sup, idx_flat, d).reshape(g * n, 4, d)
    sel = ((idx_flat & 3)[:, None] == jnp.arange(4)[None, :]).astype(jnp.float32)
    emb = jnp.einsum("bcd,bc->bd", sup_rows, sel).reshape(g, n, d)
    quantized_all = jnp.transpose(emb, (1, 0, 2))
    return (loss[0, 0], quantized_all, perp[0, 0], emb, oh, dist)


# speedup vs baseline: 1.1397x; 1.0512x over previous
import functools

import jax
import jax.numpy as jnp
from jax import lax
from jax.experimental import pallas as pl
from jax.experimental.pallas import tpu as pltpu
from jax.experimental.pallas import tpu_sc as plsc

_COMMIT = 0.25
_BN = 256


def _vq_tc_kernel(x_ref, w_ref,
                  dist_ref, oh_ref, idx_ref, wpad_ref, loss_ref, perp_ref,
                  counts_ref, ws_ref, acc_ref):
    g = pl.program_id(0)
    i = pl.program_id(1)
    ng = pl.num_programs(0)
    nb = pl.num_programs(1)

    x = x_ref[0]
    w = w_ref[0]
    bn, d = x.shape
    k = w.shape[0]

    @pl.when(jnp.logical_and(g == 0, i == 0))
    def _():
        acc_ref[0] = 0.0
        acc_ref[1] = 0.0

    @pl.when(i == 0)
    def _():
        counts_ref[...] = jnp.zeros_like(counts_ref)
        wpad_ref[0] = jnp.concatenate(
            [w, jnp.zeros((k, 128 - d), jnp.float32)], axis=1)
        w2 = w * w
        ones = jnp.ones((1, d), jnp.float32)
        ws_ref[...] = jax.lax.dot_general(
            ones, w2, (((1,), (1,)), ((), ())),
            preferred_element_type=jnp.float32)

    xs = jnp.sum(x * x, axis=1, keepdims=True)
    dots = jax.lax.dot_general(
        x, w, (((1,), (1,)), ((), ())),
        preferred_element_type=jnp.float32)
    dist = xs + ws_ref[...] - 2.0 * dots
    dist_ref[0] = dist

    mind = jnp.min(dist, axis=1, keepdims=True)
    iota = jax.lax.broadcasted_iota(jnp.int32, (bn, k), 1)
    idx = jnp.min(jnp.where(dist == mind, iota, k),
                  axis=1, keepdims=True)
    oh = (iota == idx).astype(jnp.float32)
    oh_ref[0] = oh
    idx_ref[0] = idx + g * k
    counts_ref[...] += jnp.sum(oh, axis=0, keepdims=True)

    acc_ref[0] += jnp.sum(mind)

    @pl.when(i == nb - 1)
    def _():
        n_tok = nb * bn
        p = counts_ref[...] * (1.0 / n_tok)
        ent = -jnp.sum(p * jnp.log(p + 1e-10))
        acc_ref[1] += jnp.exp(ent)

    @pl.when(jnp.logical_and(g == ng - 1, i == nb - 1))
    def _():
        n_tok = nb * bn
        loss_ref[...] = jnp.full(
            (1, 1), acc_ref[0] * ((1.0 + _COMMIT) / (ng * n_tok * d)),
            dtype=jnp.float32)
        perp_ref[...] = jnp.full((1, 1), acc_ref[1] * (1.0 / ng),
                                 dtype=jnp.float32)


def _sc_gather(table, idx_flat, d_out):
    b, = idx_flat.shape
    _, dp = table.shape
    info = plsc.get_sparse_core_info()
    nw = info.num_cores * info.num_subcores
    bpw = b // nw
    mesh = plsc.VectorSubcoreMesh(core_axis_name="c", subcore_axis_name="s")

    @functools.partial(
        pl.kernel, mesh=mesh,
        out_type=jax.ShapeDtypeStruct((b, dp), jnp.float32),
        scratch_types=[
            pltpu.VMEM((bpw,), jnp.int32),
            pltpu.VMEM((bpw, dp), jnp.float32),
            pltpu.SemaphoreType.DMA,
        ],
    )
    def k(idx_hbm, table_hbm, out_hbm, idx_v, rows_v, sem):
        wid = lax.axis_index("s") * info.num_cores + lax.axis_index("c")
        base = wid * bpw
        pltpu.sync_copy(idx_hbm.at[pl.ds(base, bpw)], idx_v)
        pltpu.async_copy(table_hbm.at[idx_v], rows_v, sem).wait()
        pltpu.sync_copy(rows_v, out_hbm.at[pl.ds(base, bpw)])

    return k(idx_flat, table)[:, :d_out]


def kernel(inputs, weights):
    n, g, d = inputs.shape
    _, k, _ = weights.shape
    bn = _BN
    x = jnp.transpose(inputs, (1, 0, 2))

    grid = (g, n // bn)
    out_shape = (
        jax.ShapeDtypeStruct((g, n, k), jnp.float32),
        jax.ShapeDtypeStruct((g, n, k), jnp.float32),
        jax.ShapeDtypeStruct((g, n, 1), jnp.int32),
        jax.ShapeDtypeStruct((g, k, 128), jnp.float32),
        jax.ShapeDtypeStruct((1, 1), jnp.float32),
        jax.ShapeDtypeStruct((1, 1), jnp.float32),
    )
    in_specs = [
        pl.BlockSpec((1, bn, d), lambda gi, ii: (gi, ii, 0)),
        pl.BlockSpec((1, k, d), lambda gi, ii: (gi, 0, 0)),
    ]
    out_specs = (
        pl.BlockSpec((1, bn, k), lambda gi, ii: (gi, ii, 0)),
        pl.BlockSpec((1, bn, k), lambda gi, ii: (gi, ii, 0)),
        pl.BlockSpec((1, bn, 1), lambda gi, ii: (gi, ii, 0)),
        pl.BlockSpec((1, k, 128), lambda gi, ii: (gi, 0, 0)),
        pl.BlockSpec((1, 1), lambda gi, ii: (0, 0)),
        pl.BlockSpec((1, 1), lambda gi, ii: (0, 0)),
    )
    scratch_shapes = [
        pltpu.VMEM((1, k), jnp.float32),
        pltpu.VMEM((1, k), jnp.float32),
        pltpu.SMEM((2,), jnp.float32),
    ]
    dist, oh, idx, wpad, loss, perp = pl.pallas_call(
        _vq_tc_kernel,
        grid=grid,
        in_specs=in_specs,
        out_specs=out_specs,
        out_shape=out_shape,
        scratch_shapes=scratch_shapes,
    )(x, weights)

    table = wpad.reshape(g * k, 128)
    idx_flat = idx.reshape(g * n)
    emb = _sc_gather(table, idx_flat, d).reshape(g, n, d)
    quantized_all = jnp.transpose(emb, (1, 0, 2))
    return (loss[0, 0], quantized_all, perp[0, 0], emb, oh, dist)
